# baseline (speedup 1.0000x reference)
import jax
import jax.numpy as jnp
from jax import lax
from jax.experimental import pallas as pl
from jax.experimental.pallas import tpu as pltpu
from jax.experimental.pallas import tpu_sc as plsc

N_COMPONENTS = 3
N_NODES = 883
N_TIMESTEPS = 12
DIM = N_COMPONENTS * N_NODES * N_TIMESTEPS
BATCH = 1024

NC = 2
NS = 16
LANES = 16
NW = NC * NS
N_SLABS = N_COMPONENTS * N_TIMESTEPS
N_ENTRIES = 24 + 7
N_PAD = 888
B_CHUNKS = 8
BW = BATCH // B_CHUNKS
S_RANGES = 4
SPR = N_SLABS // S_RANGES
NCHK = 224
N_TAILW = N_NODES - 3 * NCHK
CHUNKS = ((0, NCHK), (NCHK, NCHK), (2 * NCHK, NCHK), (3 * NCHK, N_TAILW))
PRIOR_ROWS = (NCHK, N_TAILW, NCHK, NCHK)


def _te_body(tab_hbm, hidx_hbm, didx_hbm, out_hbm,
             hc, dc, tb0, tb1, ob0, ob1, sem_t, sem_o):
    wid = lax.axis_index("s") * NC + lax.axis_index("c")
    bi = wid % B_CHUNKS
    sj = wid // B_CHUNKS
    b0 = bi * BW
    s_base = sj * SPR

    pltpu.sync_copy(hidx_hbm.at[pl.ds(b0, BW)], hc)
    pltpu.sync_copy(didx_hbm.at[pl.ds(b0, BW)], dc)

    hvecs = [hc[pl.ds(k * LANES, LANES)] * N_PAD
             for k in range(BW // LANES)]
    dvecs = [(dc[pl.ds(k * LANES, LANES)] + 24) * N_PAD
             for k in range(BW // LANES)]

    def run_nodes(tb, ob, n0, nrows):
        for k in range(0, BW // LANES, 2):
            hb0 = hvecs[k] + n0
            db0 = dvecs[k] + n0
            hb1 = hvecs[k + 1] + n0
            db1 = dvecs[k + 1] + n0

            @plsc.parallel_loop(0, nrows, unroll=2)
            def node_step(nl, hb0=hb0, db0=db0, hb1=hb1, db1=db1, k=k):
                v0 = (plsc.load_gather(tb, [hb0 + nl])
                      + plsc.load_gather(tb, [db0 + nl]))
                ob[nl, pl.ds(k * LANES, LANES)] = v0
                v1 = (plsc.load_gather(tb, [hb1 + nl])
                      + plsc.load_gather(tb, [db1 + nl]))
                ob[nl, pl.ds((k + 1) * LANES, LANES)] = v1

    def wait_out(rows):
        n0 = 3 * NCHK if rows == N_TAILW else 0
        pltpu.make_async_copy(
            ob0.at[pl.ds(0, rows), :],
            out_hbm.at[0, pl.ds(n0, rows), pl.ds(b0, BW)], sem_o).wait()

    def wait_tab():
        pltpu.make_async_copy(tab_hbm.at[0], tb0, sem_t).wait()

    def slab_work(s, tb, first):
        for c, (n0, nr) in enumerate(CHUNKS):
            ob = ob0 if c % 2 == 0 else ob1
            if (not first) or c >= 2:
                wait_out(PRIOR_ROWS[c])
            run_nodes(tb, ob, n0, nr)
            pltpu.async_copy(
                ob.at[pl.ds(0, nr), :],
                out_hbm.at[s, pl.ds(n0, nr), pl.ds(b0, BW)], sem_o)

    pltpu.async_copy(tab_hbm.at[s_base], tb0, sem_t)
    pltpu.async_copy(tab_hbm.at[s_base + 1], tb1, sem_t)
    wait_tab()
    slab_work(s_base, tb0, True)

    def pair_step(p, carry):
        s_a = s_base + 1 + 2 * p
        wait_tab()
        pltpu.async_copy(tab_hbm.at[s_a + 1], tb0, sem_t)
        slab_work(s_a, tb1, False)
        wait_tab()

        @pl.when(p < (SPR - 1) // 2 - 1)
        def _():
            pltpu.async_copy(tab_hbm.at[s_a + 2], tb1, sem_t)

        slab_work(s_a + 1, tb0, False)
        return carry

    lax.fori_loop(0, (SPR - 1) // 2, pair_step, 0)
    wait_out(NCHK)
    wait_out(N_TAILW)


@jax.jit
def kernel(H, D, h_ebd, d_ebd):
    ht = h_ebd.reshape(24, N_COMPONENTS, N_NODES, N_TIMESTEPS)
    ht = ht.transpose(1, 3, 0, 2)
    dt = d_ebd.reshape(7, N_COMPONENTS, N_NODES, N_TIMESTEPS)
    dt = dt.transpose(1, 3, 0, 2)
    tab = jnp.concatenate([ht, dt], axis=2)
    tab = jnp.pad(tab, ((0, 0), (0, 0), (0, 0), (0, N_PAD - N_NODES)))
    tab = tab.reshape(N_SLABS, N_ENTRIES * N_PAD)

    mesh = plsc.VectorSubcoreMesh(core_axis_name="c", subcore_axis_name="s")
    run = pl.kernel(
        _te_body,
        out_type=jax.ShapeDtypeStruct((N_SLABS, N_NODES, BATCH), jnp.float32),
        mesh=mesh,
        compiler_params=pltpu.CompilerParams(needs_layout_passes=False),
        scratch_types=[
            pltpu.VMEM((BW,), jnp.int32),
            pltpu.VMEM((BW,), jnp.int32),
            pltpu.VMEM((N_ENTRIES * N_PAD,), jnp.float32),
            pltpu.VMEM((N_ENTRIES * N_PAD,), jnp.float32),
            pltpu.VMEM((NCHK, BW), jnp.float32),
            pltpu.VMEM((NCHK, BW), jnp.float32),
            pltpu.SemaphoreType.DMA,
            pltpu.SemaphoreType.DMA,
        ],
    )
    out3 = run(tab, H.astype(jnp.int32), D.astype(jnp.int32))
    out4 = out3.reshape(N_COMPONENTS, N_TIMESTEPS, N_NODES, BATCH)
    return out4.transpose(3, 0, 2, 1)


# speedup vs baseline: 1.0281x; 1.0281x over previous
import jax
import jax.numpy as jnp
from jax import lax
from jax.experimental import pallas as pl
from jax.experimental.pallas import tpu as pltpu
from jax.experimental.pallas import tpu_sc as plsc

N_COMPONENTS = 3
N_NODES = 883
N_TIMESTEPS = 12
DIM = N_COMPONENTS * N_NODES * N_TIMESTEPS
BATCH = 1024

NC = 2
NS = 16
LANES = 16
NW = NC * NS
N_SLABS = N_COMPONENTS * N_TIMESTEPS
N_ENTRIES = 24 + 7
N_PAD = 888
B_CHUNKS = 8
BW = BATCH // B_CHUNKS
S_RANGES = 4
SPR = N_SLABS // S_RANGES
NCHK = 224
N_TAILW = N_NODES - 3 * NCHK
CHUNKS = ((0, NCHK), (NCHK, NCHK), (2 * NCHK, NCHK), (3 * NCHK, N_TAILW))
PRIOR_ROWS = (NCHK, N_TAILW, NCHK, NCHK)


def _te_body(tab_hbm, hidx_hbm, didx_hbm, out_hbm,
             hc, dc, tb0, tb1, ob0, ob1, sem_t, sem_o):
    wid = lax.axis_index("s") * NC + lax.axis_index("c")
    bi = wid % B_CHUNKS
    sj = wid // B_CHUNKS
    b0 = bi * BW
    s_base = sj * SPR

    pltpu.sync_copy(hidx_hbm.at[pl.ds(b0, BW)], hc)
    pltpu.sync_copy(didx_hbm.at[pl.ds(b0, BW)], dc)

    hvecs = [hc[pl.ds(k * LANES, LANES)] * N_PAD
             for k in range(BW // LANES)]
    dvecs = [(dc[pl.ds(k * LANES, LANES)] + 24) * N_PAD
             for k in range(BW // LANES)]

    def run_nodes(tb, ob, n0, nrows):
        for k in range(0, BW // LANES, 2):
            hb0 = hvecs[k] + n0
            db0 = dvecs[k] + n0
            hb1 = hvecs[k + 1] + n0
            db1 = dvecs[k + 1] + n0

            @plsc.parallel_loop(0, nrows, unroll=4)
            def node_step(nl, hb0=hb0, db0=db0, hb1=hb1, db1=db1, k=k):
                v0 = (plsc.load_gather(tb, [hb0 + nl])
                      + plsc.load_gather(tb, [db0 + nl]))
                ob[nl, pl.ds(k * LANES, LANES)] = v0
                v1 = (plsc.load_gather(tb, [hb1 + nl])
                      + plsc.load_gather(tb, [db1 + nl]))
                ob[nl, pl.ds((k + 1) * LANES, LANES)] = v1

    def wait_out(rows):
        n0 = 3 * NCHK if rows == N_TAILW else 0
        pltpu.make_async_copy(
            ob0.at[pl.ds(0, rows), :],
            out_hbm.at[0, pl.ds(n0, rows), pl.ds(b0, BW)], sem_o).wait()

    def wait_tab():
        pltpu.make_async_copy(tab_hbm.at[0], tb0, sem_t).wait()

    def slab_work(s, tb, first):
        for c, (n0, nr) in enumerate(CHUNKS):
            ob = ob0 if c % 2 == 0 else ob1
            if (not first) or c >= 2:
                wait_out(PRIOR_ROWS[c])
            run_nodes(tb, ob, n0, nr)
            pltpu.async_copy(
                ob.at[pl.ds(0, nr), :],
                out_hbm.at[s, pl.ds(n0, nr), pl.ds(b0, BW)], sem_o)

    pltpu.async_copy(tab_hbm.at[s_base], tb0, sem_t)
    pltpu.async_copy(tab_hbm.at[s_base + 1], tb1, sem_t)
    wait_tab()
    slab_work(s_base, tb0, True)

    def pair_step(p, carry):
        s_a = s_base + 1 + 2 * p
        wait_tab()
        pltpu.async_copy(tab_hbm.at[s_a + 1], tb0, sem_t)
        slab_work(s_a, tb1, False)
        wait_tab()

        @pl.when(p < (SPR - 1) // 2 - 1)
        def _():
            pltpu.async_copy(tab_hbm.at[s_a + 2], tb1, sem_t)

        slab_work(s_a + 1, tb0, False)
        return carry

    lax.fori_loop(0, (SPR - 1) // 2, pair_step, 0)
    wait_out(NCHK)
    wait_out(N_TAILW)


@jax.jit
def kernel(H, D, h_ebd, d_ebd):
    ht = h_ebd.reshape(24, N_COMPONENTS, N_NODES, N_TIMESTEPS)
    ht = ht.transpose(1, 3, 0, 2)
    dt = d_ebd.reshape(7, N_COMPONENTS, N_NODES, N_TIMESTEPS)
    dt = dt.transpose(1, 3, 0, 2)
    tab = jnp.concatenate([ht, dt], axis=2)
    tab = jnp.pad(tab, ((0, 0), (0, 0), (0, 0), (0, N_PAD - N_NODES)))
    tab = tab.reshape(N_SLABS, N_ENTRIES * N_PAD)

    mesh = plsc.VectorSubcoreMesh(core_axis_name="c", subcore_axis_name="s")
    run = pl.kernel(
        _te_body,
        out_type=jax.ShapeDtypeStruct((N_SLABS, N_NODES, BATCH), jnp.float32),
        mesh=mesh,
        compiler_params=pltpu.CompilerParams(needs_layout_passes=False),
        scratch_types=[
            pltpu.VMEM((BW,), jnp.int32),
            pltpu.VMEM((BW,), jnp.int32),
            pltpu.VMEM((N_ENTRIES * N_PAD,), jnp.float32),
            pltpu.VMEM((N_ENTRIES * N_PAD,), jnp.float32),
            pltpu.VMEM((NCHK, BW), jnp.float32),
            pltpu.VMEM((NCHK, BW), jnp.float32),
            pltpu.SemaphoreType.DMA,
            pltpu.SemaphoreType.DMA,
        ],
    )
    out3 = run(tab, H.astype(jnp.int32), D.astype(jnp.int32))
    out4 = out3.reshape(N_COMPONENTS, N_TIMESTEPS, N_NODES, BATCH)
    return out4.transpose(3, 0, 2, 1)


# speedup vs baseline: 1.0779x; 1.0484x over previous
import jax
import jax.numpy as jnp
from jax import lax
from jax.experimental import pallas as pl
from jax.experimental.pallas import tpu as pltpu
from jax.experimental.pallas import tpu_sc as plsc

N_COMPONENTS = 3
N_NODES = 883
N_TIMESTEPS = 12
DIM = N_COMPONENTS * N_NODES * N_TIMESTEPS
BATCH = 1024

NC = 2
NS = 16
LANES = 16
NW = NC * NS
N_SLABS = N_COMPONENTS * N_TIMESTEPS
N_ENTRIES = 24 + 7
N_PAD = 888
B_CHUNKS = 8
BW = BATCH // B_CHUNKS
S_RANGES = 4
SPR = N_SLABS // S_RANGES
NCHK = 224
N_TAILW = N_NODES - 3 * NCHK
CHUNKS = ((0, NCHK), (NCHK, NCHK), (2 * NCHK, NCHK), (3 * NCHK, N_TAILW))
PRIOR_ROWS = (NCHK, N_TAILW, NCHK, NCHK)


def _te_body(tab_hbm, hidx_hbm, didx_hbm, out_hbm,
             hc, dc, tb0, tb1, ob0, ob1, sem_t, sem_o):
    wid = lax.axis_index("s") * NC + lax.axis_index("c")
    bi = wid % B_CHUNKS
    sj = wid // B_CHUNKS
    b0 = bi * BW
    s_base = sj * SPR

    pltpu.sync_copy(hidx_hbm.at[pl.ds(b0, BW)], hc)
    pltpu.sync_copy(didx_hbm.at[pl.ds(b0, BW)], dc)

    hvecs = [hc[pl.ds(k * LANES, LANES)] * N_PAD
             for k in range(BW // LANES)]
    dvecs = [(dc[pl.ds(k * LANES, LANES)] + 24) * N_PAD
             for k in range(BW // LANES)]

    def run_nodes(tb, ob, n0, nrows):
        for k in range(BW // LANES):
            hb = hvecs[k] + n0
            db = dvecs[k] + n0

            @plsc.parallel_loop(0, nrows, unroll=4)
            def node_step(nl, hb=hb, db=db, k=k):
                v = (plsc.load_gather(tb, [hb + nl])
                     + plsc.load_gather(tb, [db + nl]))
                ob[nl, pl.ds(k * LANES, LANES)] = v

    def wait_out(rows):
        n0 = 3 * NCHK if rows == N_TAILW else 0
        pltpu.make_async_copy(
            ob0.at[pl.ds(0, rows), :],
            out_hbm.at[0, pl.ds(n0, rows), pl.ds(b0, BW)], sem_o).wait()

    def wait_tab():
        pltpu.make_async_copy(tab_hbm.at[0], tb0, sem_t).wait()

    def slab_work(s, tb, first):
        for c, (n0, nr) in enumerate(CHUNKS):
            ob = ob0 if c % 2 == 0 else ob1
            if (not first) or c >= 2:
                wait_out(PRIOR_ROWS[c])
            run_nodes(tb, ob, n0, nr)
            pltpu.async_copy(
                ob.at[pl.ds(0, nr), :],
                out_hbm.at[s, pl.ds(n0, nr), pl.ds(b0, BW)], sem_o)

    pltpu.async_copy(tab_hbm.at[s_base], tb0, sem_t)
    pltpu.async_copy(tab_hbm.at[s_base + 1], tb1, sem_t)
    wait_tab()
    slab_work(s_base, tb0, True)

    def pair_step(p, carry):
        s_a = s_base + 1 + 2 * p
        wait_tab()
        pltpu.async_copy(tab_hbm.at[s_a + 1], tb0, sem_t)
        slab_work(s_a, tb1, False)
        wait_tab()

        @pl.when(p < (SPR - 1) // 2 - 1)
        def _():
            pltpu.async_copy(tab_hbm.at[s_a + 2], tb1, sem_t)

        slab_work(s_a + 1, tb0, False)
        return carry

    lax.fori_loop(0, (SPR - 1) // 2, pair_step, 0)
    wait_out(NCHK)
    wait_out(N_TAILW)


@jax.jit
def kernel(H, D, h_ebd, d_ebd):
    ht = h_ebd.reshape(24, N_COMPONENTS, N_NODES, N_TIMESTEPS)
    ht = ht.transpose(1, 3, 0, 2)
    dt = d_ebd.reshape(7, N_COMPONENTS, N_NODES, N_TIMESTEPS)
    dt = dt.transpose(1, 3, 0, 2)
    tab = jnp.concatenate([ht, dt], axis=2)
    tab = jnp.pad(tab, ((0, 0), (0, 0), (0, 0), (0, N_PAD - N_NODES)))
    tab = tab.reshape(N_SLABS, N_ENTRIES * N_PAD)

    mesh = plsc.VectorSubcoreMesh(core_axis_name="c", subcore_axis_name="s")
    run = pl.kernel(
        _te_body,
        out_type=jax.ShapeDtypeStruct((N_SLABS, N_NODES, BATCH), jnp.float32),
        mesh=mesh,
        compiler_params=pltpu.CompilerParams(needs_layout_passes=False),
        scratch_types=[
            pltpu.VMEM((BW,), jnp.int32),
            pltpu.VMEM((BW,), jnp.int32),
            pltpu.VMEM((N_ENTRIES * N_PAD,), jnp.float32),
            pltpu.VMEM((N_ENTRIES * N_PAD,), jnp.float32),
            pltpu.VMEM((NCHK, BW), jnp.float32),
            pltpu.VMEM((NCHK, BW), jnp.float32),
            pltpu.SemaphoreType.DMA,
            pltpu.SemaphoreType.DMA,
        ],
    )
    out3 = run(tab, H.astype(jnp.int32), D.astype(jnp.int32))
    out4 = out3.reshape(N_COMPONENTS, N_TIMESTEPS, N_NODES, BATCH)
    return out4.transpose(3, 0, 2, 1)


# speedup vs baseline: 1.1440x; 1.0613x over previous
import jax
import jax.numpy as jnp
from jax import lax
from jax.experimental import pallas as pl
from jax.experimental.pallas import tpu as pltpu
from jax.experimental.pallas import tpu_sc as plsc

N_COMPONENTS = 3
N_NODES = 883
N_TIMESTEPS = 12
DIM = N_COMPONENTS * N_NODES * N_TIMESTEPS
BATCH = 1024

NC = 2
NS = 16
LANES = 16
NW = NC * NS
N_SLABS = N_COMPONENTS * N_TIMESTEPS
N_ENTRIES = 24 + 7
N_PAD = 888
B_CHUNKS = 8
BW = BATCH // B_CHUNKS
S_RANGES = 4
SPR = N_SLABS // S_RANGES
NCHK = 224
N_TAILW = N_NODES - 3 * NCHK
CHUNKS = ((0, NCHK), (NCHK, NCHK), (2 * NCHK, NCHK), (3 * NCHK, N_TAILW))
PRIOR_ROWS = (NCHK, N_TAILW, NCHK, NCHK)


def _te_body(tab_hbm, hidx_hbm, didx_hbm, out_hbm,
             hc, dc, tb0, tb1, ob0, ob1, sem_t, sem_o):
    wid = lax.axis_index("s") * NC + lax.axis_index("c")
    bi = wid % B_CHUNKS
    sj = wid // B_CHUNKS
    b0 = bi * BW
    s_base = sj * SPR

    pltpu.sync_copy(hidx_hbm.at[pl.ds(b0, BW)], hc)
    pltpu.sync_copy(didx_hbm.at[pl.ds(b0, BW)], dc)

    hvecs = [hc[pl.ds(k * LANES, LANES)] * N_PAD
             for k in range(BW // LANES)]
    dvecs = [(dc[pl.ds(k * LANES, LANES)] + 24) * N_PAD
             for k in range(BW // LANES)]

    def run_nodes(tb, ob, n0, nrows):
        for k in range(BW // LANES):
            hb = hvecs[k] + n0
            db = dvecs[k] + n0

            @plsc.parallel_loop(0, nrows, unroll=4)
            def node_step(nl, hb=hb, db=db, k=k):
                v = (plsc.load_gather(tb, [hb + nl])
                     + plsc.load_gather(tb, [db + nl]))
                ob[nl, pl.ds(k * LANES, LANES)] = v

    def wait_out(rows):
        n0 = 3 * NCHK if rows == N_TAILW else 0
        pltpu.make_async_copy(
            ob0.at[pl.ds(0, rows), :],
            out_hbm.at[0, pl.ds(n0, rows), pl.ds(b0, BW)], sem_o).wait()

    def wait_tab():
        pltpu.make_async_copy(tab_hbm.at[0], tb0, sem_t).wait()

    def slab_work(s, tb, first):
        for c, (n0, nr) in enumerate(CHUNKS):
            ob = ob0 if c % 2 == 0 else ob1
            if (not first) or c >= 2:
                wait_out(PRIOR_ROWS[c])
            run_nodes(tb, ob, n0, nr)
            pltpu.async_copy(
                ob.at[pl.ds(0, nr), :],
                out_hbm.at[s, pl.ds(n0, nr), pl.ds(b0, BW)], sem_o)

    pltpu.async_copy(tab_hbm.at[s_base], tb0, sem_t)
    pltpu.async_copy(tab_hbm.at[s_base + 1], tb1, sem_t)
    wait_tab()
    slab_work(s_base, tb0, True)

    def pair_step(p, carry):
        s_a = s_base + 1 + 2 * p
        wait_tab()
        pltpu.async_copy(tab_hbm.at[s_a + 1], tb0, sem_t)
        slab_work(s_a, tb1, False)
        wait_tab()

        @pl.when(p < (SPR - 1) // 2 - 1)
        def _():
            pltpu.async_copy(tab_hbm.at[s_a + 2], tb1, sem_t)

        slab_work(s_a + 1, tb0, False)
        return carry

    lax.fori_loop(0, (SPR - 1) // 2, pair_step, 0)
    wait_out(NCHK)
    wait_out(N_TAILW)


@jax.jit
def kernel(H, D, h_ebd, d_ebd):
    tab = jnp.concatenate([h_ebd, d_ebd], axis=0)
    tab = tab.reshape(N_ENTRIES, N_COMPONENTS, N_NODES, N_TIMESTEPS)
    tab = tab.transpose(1, 3, 0, 2)
    tab = jnp.pad(tab, ((0, 0), (0, 0), (0, 0), (0, N_PAD - N_NODES)))
    tab = tab.reshape(N_SLABS, N_ENTRIES * N_PAD)

    mesh = plsc.VectorSubcoreMesh(core_axis_name="c", subcore_axis_name="s")
    run = pl.kernel(
        _te_body,
        out_type=jax.ShapeDtypeStruct((N_SLABS, N_NODES, BATCH), jnp.float32),
        mesh=mesh,
        compiler_params=pltpu.CompilerParams(needs_layout_passes=False),
        scratch_types=[
            pltpu.VMEM((BW,), jnp.int32),
            pltpu.VMEM((BW,), jnp.int32),
            pltpu.VMEM((N_ENTRIES * N_PAD,), jnp.float32),
            pltpu.VMEM((N_ENTRIES * N_PAD,), jnp.float32),
            pltpu.VMEM((NCHK, BW), jnp.float32),
            pltpu.VMEM((NCHK, BW), jnp.float32),
            pltpu.SemaphoreType.DMA,
            pltpu.SemaphoreType.DMA,
        ],
    )
    out3 = run(tab, H.astype(jnp.int32), D.astype(jnp.int32))
    out4 = out3.reshape(N_COMPONENTS, N_TIMESTEPS, N_NODES, BATCH)
    return out4.transpose(3, 0, 2, 1)
